# baseline (speedup 1.0000x reference)
import functools

import jax
import jax.numpy as jnp
from jax import lax
from jax.experimental import pallas as pl
from jax.experimental.pallas import tpu as pltpu
from jax.experimental.pallas import tpu_sc as plsc

_B = 1024
_NNEG = 50
_DIM = 64
_NPAIR = 6
_NHEAD = 5

_NP = 56
_PAIRB = 2
_NPP = _NP * _PAIRB
_BP = _B // _PAIRB
_NC = 2
_NS = 16
_NW = _NC * _NS
_IPW = _BP // _NW


def _gates_tc(x, gw, gb):
    def body(x_ref, w_ref, b_ref, o_ref):
        o_ref[...] = jnp.tanh(
            jnp.dot(x_ref[...], w_ref[...], preferred_element_type=jnp.float32,
                    precision=lax.Precision.HIGHEST)
            + b_ref[...]
        )
    return pl.pallas_call(
        body,
        out_shape=jax.ShapeDtypeStruct((_B, _NPAIR * _DIM), jnp.float32),
    )(x, gw, gb)


def _reduce_tc(partials):
    def body(p_ref, o_ref):
        r = lax.broadcasted_iota(jnp.int32, (_NPP * 16, _NPP), 0)
        c = lax.broadcasted_iota(jnp.int32, (_NPP * 16, _NPP), 1)
        sel = jnp.where(r // 16 == c, 1.0, 0.0).astype(jnp.float32)
        o_ref[...] = jnp.dot(p_ref[...], sel,
                             preferred_element_type=jnp.float32,
                             precision=lax.Precision.HIGHEST)
    return pl.pallas_call(
        body,
        out_shape=jax.ShapeDtypeStruct((_BP, _NPP), jnp.float32),
    )(partials)


def _make_sc_kernel():
    mesh = plsc.VectorSubcoreMesh(core_axis_name="c", subcore_axis_name="s")

    @functools.partial(
        pl.kernel,
        out_type=jax.ShapeDtypeStruct((_BP, _NPP, 16), jnp.float32),
        mesh=mesh,
        scratch_types=[
            pltpu.VMEM((2, _NHEAD, _NPP), jnp.int32),
            pltpu.VMEM((2, _PAIRB, _NPAIR, _DIM), jnp.float32),
            pltpu.VMEM((2, _NHEAD, _NPP, _DIM), jnp.float32),
            pltpu.VMEM((2, _NPP, 16), jnp.float32),
            pltpu.SemaphoreType.DMA,
            pltpu.SemaphoreType.DMA,
            pltpu.SemaphoreType.DMA,
            pltpu.SemaphoreType.DMA,
        ],
        compiler_params=pltpu.CompilerParams(
            needs_layout_passes=False, use_tc_tiling_on_sc=False),
    )
    def sc(labels_hbm, gates_hbm, t0, t1, t2, t3, t4, out_hbm,
           lab_v, gate_v, rows_v, accs_v, sem_g0, sem_g1, sem_o0, sem_o1):
        wid = lax.axis_index("s") * _NC + lax.axis_index("c")
        base = wid * _IPW
        tables = (t0, t1, t2, t3, t4)
        sem_g = (sem_g0, sem_g1)
        sem_o = (sem_o0, sem_o1)

        def prefetch(p, s):
            pltpu.sync_copy(labels_hbm.at[p], lab_v.at[s])
            pltpu.async_copy(gates_hbm.at[p], gate_v.at[s], sem_g[s])
            for h in range(_NHEAD):
                pltpu.async_copy(tables[h].at[lab_v.at[s, h]],
                                 rows_v.at[s, h], sem_g[s])

        def drain_g(s):
            pltpu.make_async_copy(gates_hbm.at[0], gate_v.at[s],
                                  sem_g[s]).wait()
            for h in range(_NHEAD):
                pltpu.make_async_copy(out_hbm.at[0], rows_v.at[s, h],
                                      sem_g[s]).wait()

        def wait_o(s):
            pltpu.make_async_copy(accs_v.at[s], out_hbm.at[base + s],
                                  sem_o[s]).wait()

        def consume(p, s):
            drain_g(s)
            wait_o(s)
            for half in range(_PAIRB):
                g = [[gate_v[s, half, pr, pl.ds(c * 16, 16)]
                      for c in range(4)] for pr in range(_NPAIR)]

                def nbody(n, _, half=half, g=g, s=s):
                    m = half * _NP + n
                    acc = None
                    for c in range(4):
                        sl = pl.ds(c * 16, 16)
                        pred = rows_v[s, 0, m, sl]
                        perm = rows_v[s, 1, m, sl]
                        prim = rows_v[s, 2, m, sl]
                        sec = rows_v[s, 3, m, sl]
                        reo = rows_v[s, 4, m, sl]
                        a1 = pred * g[0][c] + reo * g[3][c]
                        a2 = pred * g[1][c] + reo * g[4][c]
                        a3 = pred * g[2][c] + reo * g[5][c]
                        contrib = a1 * perm + a2 * prim + a3 * sec
                        acc = contrib if c == 0 else acc + contrib
                    accs_v[s, m] = acc
                    return 0

                lax.fori_loop(0, _NP, nbody, 0)
            pltpu.async_copy(accs_v.at[s], out_hbm.at[p], sem_o[s])

        pltpu.async_copy(accs_v.at[0], out_hbm.at[base + 0], sem_o0)
        pltpu.async_copy(accs_v.at[1], out_hbm.at[base + 1], sem_o1)
        prefetch(base + 0, 0)

        def body(k, carry):
            i0 = base + 2 * k
            prefetch(i0 + 1, 1)
            consume(i0, 0)
            nxt = jnp.minimum(i0 + 2, base + _IPW - 1)
            prefetch(nxt, 0)
            consume(i0 + 1, 1)
            return carry

        lax.fori_loop(0, _IPW // 2, body, 0)

        drain_g(0)
        wait_o(0)
        wait_o(1)

    return sc


_sc_kernel = _make_sc_kernel()


def kernel(x, neg_labels, emb_predictor, emb_cf_perm, emb_cf_primary,
           emb_cf_secondary, emb_reorder,
           gw_predictor__cf_perm, gb_predictor__cf_perm,
           gw_predictor__cf_primary, gb_predictor__cf_primary,
           gw_predictor__cf_secondary, gb_predictor__cf_secondary,
           gw_reorder__cf_perm, gb_reorder__cf_perm,
           gw_reorder__cf_primary, gb_reorder__cf_primary,
           gw_reorder__cf_secondary, gb_reorder__cf_secondary):
    gw = jnp.concatenate(
        [gw_predictor__cf_perm, gw_predictor__cf_primary,
         gw_predictor__cf_secondary, gw_reorder__cf_perm,
         gw_reorder__cf_primary, gw_reorder__cf_secondary], axis=1)
    gb = jnp.concatenate(
        [gb_predictor__cf_perm, gb_predictor__cf_primary,
         gb_predictor__cf_secondary, gb_reorder__cf_perm,
         gb_reorder__cf_primary, gb_reorder__cf_secondary], axis=0)
    gates = _gates_tc(x, gw, gb.reshape(1, _NPAIR * _DIM))
    gates = gates.reshape(_BP, _PAIRB, _NPAIR, _DIM)

    lab = jnp.transpose(neg_labels[:, :, :_NHEAD], (0, 2, 1))
    lab = jnp.pad(lab, ((0, 0), (0, 0), (0, _NP - _NNEG)))
    lab = lab.reshape(_BP, _PAIRB, _NHEAD, _NP)
    lab = jnp.transpose(lab, (0, 2, 1, 3)).reshape(_BP, _NHEAD, _NPP)

    partials = _sc_kernel(lab, gates, emb_predictor, emb_cf_perm,
                          emb_cf_primary, emb_cf_secondary, emb_reorder)
    score = _reduce_tc(partials.reshape(_BP, _NPP * 16))
    score = score.reshape(_B, _NP)
    return score[:, :_NNEG]


# speedup vs baseline: 1.2073x; 1.2073x over previous
import functools

import jax
import jax.numpy as jnp
from jax import lax
from jax.experimental import pallas as pl
from jax.experimental.pallas import tpu as pltpu
from jax.experimental.pallas import tpu_sc as plsc

_B = 1024
_NNEG = 50
_DIM = 64
_IN_DIM = 128
_NPAIR = 6
_NHEAD = 5

_NP = 56
_NC = 2
_NS = 16
_NW = _NC * _NS
_BPW = _B // _NW


def _gates_tc(x, gw, gb):
    def body(x_ref, w_ref, b_ref, o_ref):
        o_ref[...] = jnp.tanh(
            jnp.dot(x_ref[...], w_ref[...], preferred_element_type=jnp.float32,
                    precision=lax.Precision.HIGHEST)
            + b_ref[...]
        )
    return pl.pallas_call(
        body,
        out_shape=jax.ShapeDtypeStruct((_B, _NPAIR * _DIM), jnp.float32),
    )(x, gw, gb)


def _reduce_tc(partials):
    def body(p_ref, o_ref):
        r = lax.broadcasted_iota(jnp.int32, (_NP * 16, _NP), 0)
        c = lax.broadcasted_iota(jnp.int32, (_NP * 16, _NP), 1)
        sel = jnp.where(r // 16 == c, 1.0, 0.0).astype(jnp.float32)
        o_ref[...] = jnp.dot(p_ref[...], sel,
                             preferred_element_type=jnp.float32,
                             precision=lax.Precision.HIGHEST)
    return pl.pallas_call(
        body,
        out_shape=jax.ShapeDtypeStruct((_B, _NP), jnp.float32),
    )(partials)


def _make_sc_kernel():
    mesh = plsc.VectorSubcoreMesh(core_axis_name="c", subcore_axis_name="s")

    @functools.partial(
        pl.kernel,
        out_type=jax.ShapeDtypeStruct((_B, _NP, 16), jnp.float32),
        mesh=mesh,
        scratch_types=[
            pltpu.VMEM((_NHEAD, _NP), jnp.int32),
            pltpu.VMEM((_NPAIR, _DIM), jnp.float32),
            pltpu.VMEM((_NHEAD, _NP, _DIM), jnp.float32),
            pltpu.VMEM((_NP, 16), jnp.float32),
            pltpu.SemaphoreType.DMA,
        ],
        compiler_params=pltpu.CompilerParams(
            needs_layout_passes=False, use_tc_tiling_on_sc=False),
    )
    def sc(labels_hbm, gates_hbm, t0, t1, t2, t3, t4, out_hbm,
           lab_v, gate_v, rows_v, accs_v, sem):
        wid = lax.axis_index("s") * _NC + lax.axis_index("c")
        tables = (t0, t1, t2, t3, t4)

        def batch_body(i, carry):
            b = wid * _BPW + i
            pltpu.sync_copy(labels_hbm.at[b], lab_v)
            pltpu.sync_copy(gates_hbm.at[b], gate_v)
            cps = [
                pltpu.async_copy(tables[h].at[lab_v.at[h]], rows_v.at[h], sem)
                for h in range(_NHEAD)
            ]
            for cp in cps:
                cp.wait()

            for c in range(4):
                sl = pl.ds(c * 16, 16)
                g0 = gate_v[0, sl]
                g1 = gate_v[1, sl]
                g2 = gate_v[2, sl]
                g3 = gate_v[3, sl]
                g4 = gate_v[4, sl]
                g5 = gate_v[5, sl]

                def neg_body(n, _, c=c, sl=sl, g0=g0, g1=g1, g2=g2,
                             g3=g3, g4=g4, g5=g5):
                    pred = rows_v[0, n, sl]
                    perm = rows_v[1, n, sl]
                    prim = rows_v[2, n, sl]
                    sec = rows_v[3, n, sl]
                    reo = rows_v[4, n, sl]
                    a1 = pred * g0 + reo * g3
                    a2 = pred * g1 + reo * g4
                    a3 = pred * g2 + reo * g5
                    contrib = a1 * perm + a2 * prim + a3 * sec
                    if c == 0:
                        accs_v[n] = contrib
                    else:
                        plsc.addupdate(accs_v.at[n], contrib)
                    return 0

                lax.fori_loop(0, _NP, neg_body, 0)

            pltpu.sync_copy(accs_v, out_hbm.at[b])
            return carry

        lax.fori_loop(0, _BPW, batch_body, 0)

    return sc


_sc_kernel = _make_sc_kernel()


def kernel(x, neg_labels, emb_predictor, emb_cf_perm, emb_cf_primary,
           emb_cf_secondary, emb_reorder,
           gw_predictor__cf_perm, gb_predictor__cf_perm,
           gw_predictor__cf_primary, gb_predictor__cf_primary,
           gw_predictor__cf_secondary, gb_predictor__cf_secondary,
           gw_reorder__cf_perm, gb_reorder__cf_perm,
           gw_reorder__cf_primary, gb_reorder__cf_primary,
           gw_reorder__cf_secondary, gb_reorder__cf_secondary):
    gw = jnp.concatenate(
        [gw_predictor__cf_perm, gw_predictor__cf_primary,
         gw_predictor__cf_secondary, gw_reorder__cf_perm,
         gw_reorder__cf_primary, gw_reorder__cf_secondary], axis=1)
    gb = jnp.concatenate(
        [gb_predictor__cf_perm, gb_predictor__cf_primary,
         gb_predictor__cf_secondary, gb_reorder__cf_perm,
         gb_reorder__cf_primary, gb_reorder__cf_secondary], axis=0)
    gates = _gates_tc(x, gw, gb.reshape(1, _NPAIR * _DIM))
    gates = gates.reshape(_B, _NPAIR, _DIM)

    lab = jnp.transpose(neg_labels[:, :, :_NHEAD], (0, 2, 1))
    lab = jnp.pad(lab, ((0, 0), (0, 0), (0, _NP - _NNEG)))

    partials = _sc_kernel(lab, gates, emb_predictor, emb_cf_perm,
                          emb_cf_primary, emb_cf_secondary, emb_reorder)
    score = _reduce_tc(partials.reshape(_B, _NP * 16))
    return score[:, :_NNEG]


# speedup vs baseline: 1.2106x; 1.0027x over previous
import functools

import jax
import jax.numpy as jnp
from jax import lax
from jax.experimental import pallas as pl
from jax.experimental.pallas import tpu as pltpu
from jax.experimental.pallas import tpu_sc as plsc

_B = 1024
_NNEG = 50
_DIM = 64
_NPAIR = 6
_NHEAD = 5

_NP = 56
_NC = 2
_NS = 16
_NW = _NC * _NS
_BPW = _B // _NW


def _gates_tc(x, gw, gb):
    def body(x_ref, w_ref, b_ref, o_ref):
        o_ref[...] = jnp.tanh(
            jnp.dot(x_ref[...], w_ref[...], preferred_element_type=jnp.float32,
                    precision=lax.Precision.HIGHEST)
            + b_ref[...]
        )
    return pl.pallas_call(
        body,
        out_shape=jax.ShapeDtypeStruct((_B, _NPAIR * _DIM), jnp.float32),
    )(x, gw, gb)


def _reduce_tc(partials):
    def body(p_ref, o_ref):
        r = lax.broadcasted_iota(jnp.int32, (_NP * 16, _NP), 0)
        c = lax.broadcasted_iota(jnp.int32, (_NP * 16, _NP), 1)
        sel = jnp.where(r // 16 == c, 1.0, 0.0).astype(jnp.float32)
        o_ref[...] = jnp.dot(p_ref[...], sel,
                             preferred_element_type=jnp.float32,
                             precision=lax.Precision.HIGHEST)
    return pl.pallas_call(
        body,
        out_shape=jax.ShapeDtypeStruct((_B, _NP), jnp.float32),
    )(partials)


def _make_sc_kernel():
    mesh = plsc.VectorSubcoreMesh(core_axis_name="c", subcore_axis_name="s")

    @functools.partial(
        pl.kernel,
        out_type=jax.ShapeDtypeStruct((_B, _NP, 16), jnp.float32),
        mesh=mesh,
        scratch_types=[
            pltpu.VMEM((2, _NHEAD, _NP), jnp.int32),
            pltpu.VMEM((2, _NPAIR, _DIM), jnp.float32),
            pltpu.VMEM((2, _NHEAD, _NP, _DIM), jnp.float32),
            pltpu.VMEM((2, _NP, 16), jnp.float32),
            pltpu.SemaphoreType.DMA,
            pltpu.SemaphoreType.DMA,
            pltpu.SemaphoreType.DMA,
            pltpu.SemaphoreType.DMA,
        ],
        compiler_params=pltpu.CompilerParams(
            needs_layout_passes=False, use_tc_tiling_on_sc=False),
    )
    def sc(labels_hbm, gates_hbm, t0, t1, t2, t3, t4, out_hbm,
           lab_v, gate_v, rows_v, accs_v, sem_g0, sem_g1, sem_o0, sem_o1):
        wid = lax.axis_index("s") * _NC + lax.axis_index("c")
        base = wid * _BPW
        last = base + _BPW - 1
        tables = (t0, t1, t2, t3, t4)
        sem_g = (sem_g0, sem_g1)
        sem_o = (sem_o0, sem_o1)

        def prefetch(b, s):
            pltpu.sync_copy(labels_hbm.at[b], lab_v.at[s])
            pltpu.async_copy(gates_hbm.at[b], gate_v.at[s], sem_g[s])
            for h in range(_NHEAD):
                pltpu.async_copy(tables[h].at[lab_v.at[s, h]],
                                 rows_v.at[s, h], sem_g[s])

        def drain_g(s):
            pltpu.make_async_copy(gates_hbm.at[0], gate_v.at[s],
                                  sem_g[s]).wait()
            for h in range(_NHEAD):
                pltpu.make_async_copy(out_hbm.at[0], rows_v.at[s, h],
                                      sem_g[s]).wait()

        def wait_o(s):
            pltpu.make_async_copy(accs_v.at[s], out_hbm.at[base + s],
                                  sem_o[s]).wait()

        def consume(b, s):
            drain_g(s)
            wait_o(s)
            for c in range(4):
                sl = pl.ds(c * 16, 16)
                g0 = gate_v[s, 0, sl]
                g1 = gate_v[s, 1, sl]
                g2 = gate_v[s, 2, sl]
                g3 = gate_v[s, 3, sl]
                g4 = gate_v[s, 4, sl]
                g5 = gate_v[s, 5, sl]

                def neg_body(n, _, c=c, sl=sl, s=s, g0=g0, g1=g1, g2=g2,
                             g3=g3, g4=g4, g5=g5):
                    pred = rows_v[s, 0, n, sl]
                    perm = rows_v[s, 1, n, sl]
                    prim = rows_v[s, 2, n, sl]
                    sec = rows_v[s, 3, n, sl]
                    reo = rows_v[s, 4, n, sl]
                    a1 = pred * g0 + reo * g3
                    a2 = pred * g1 + reo * g4
                    a3 = pred * g2 + reo * g5
                    contrib = a1 * perm + a2 * prim + a3 * sec
                    if c == 0:
                        accs_v[s, n] = contrib
                    else:
                        plsc.addupdate(accs_v.at[s, n], contrib)
                    return 0

                lax.fori_loop(0, _NP, neg_body, 0)

            pltpu.async_copy(accs_v.at[s], out_hbm.at[b], sem_o[s])

        pltpu.async_copy(accs_v.at[0], out_hbm.at[base + 0], sem_o0)
        pltpu.async_copy(accs_v.at[1], out_hbm.at[base + 1], sem_o1)
        prefetch(base, 0)

        def body(k, carry):
            i0 = base + 2 * k
            prefetch(i0 + 1, 1)
            consume(i0, 0)
            prefetch(jnp.minimum(i0 + 2, last), 0)
            consume(i0 + 1, 1)
            return carry

        lax.fori_loop(0, _BPW // 2, body, 0)

        drain_g(0)
        wait_o(0)
        wait_o(1)

    return sc


_sc_kernel = _make_sc_kernel()


def kernel(x, neg_labels, emb_predictor, emb_cf_perm, emb_cf_primary,
           emb_cf_secondary, emb_reorder,
           gw_predictor__cf_perm, gb_predictor__cf_perm,
           gw_predictor__cf_primary, gb_predictor__cf_primary,
           gw_predictor__cf_secondary, gb_predictor__cf_secondary,
           gw_reorder__cf_perm, gb_reorder__cf_perm,
           gw_reorder__cf_primary, gb_reorder__cf_primary,
           gw_reorder__cf_secondary, gb_reorder__cf_secondary):
    gw = jnp.concatenate(
        [gw_predictor__cf_perm, gw_predictor__cf_primary,
         gw_predictor__cf_secondary, gw_reorder__cf_perm,
         gw_reorder__cf_primary, gw_reorder__cf_secondary], axis=1)
    gb = jnp.concatenate(
        [gb_predictor__cf_perm, gb_predictor__cf_primary,
         gb_predictor__cf_secondary, gb_reorder__cf_perm,
         gb_reorder__cf_primary, gb_reorder__cf_secondary], axis=0)
    gates = _gates_tc(x, gw, gb.reshape(1, _NPAIR * _DIM))
    gates = gates.reshape(_B, _NPAIR, _DIM)

    lab = jnp.transpose(neg_labels[:, :, :_NHEAD], (0, 2, 1))
    lab = jnp.pad(lab, ((0, 0), (0, 0), (0, _NP - _NNEG)))

    partials = _sc_kernel(lab, gates, emb_predictor, emb_cf_perm,
                          emb_cf_primary, emb_cf_secondary, emb_reorder)
    score = _reduce_tc(partials.reshape(_B, _NP * 16))
    return score[:, :_NNEG]


# speedup vs baseline: 1.3360x; 1.1036x over previous
import functools

import jax
import jax.numpy as jnp
from jax import lax
from jax.experimental import pallas as pl
from jax.experimental.pallas import tpu as pltpu
from jax.experimental.pallas import tpu_sc as plsc

_B = 1024
_NNEG = 50
_DIM = 64
_NPAIR = 6
_NHEAD = 5

_NP = 56
_NC = 2
_NS = 16
_NW = _NC * _NS
_BPW = _B // _NW


def _gates_tc(x, gw, gb):
    def body(x_ref, w_ref, b_ref, o_ref):
        o_ref[...] = jnp.tanh(
            jnp.dot(x_ref[...], w_ref[...], preferred_element_type=jnp.float32,
                    precision=lax.Precision.HIGHEST)
            + b_ref[...]
        )
    return pl.pallas_call(
        body,
        out_shape=jax.ShapeDtypeStruct((_B, _NPAIR * _DIM), jnp.float32),
    )(x, gw, gb)


def _make_sc_kernel():
    mesh = plsc.VectorSubcoreMesh(core_axis_name="c", subcore_axis_name="s")

    @functools.partial(
        pl.kernel,
        out_type=jax.ShapeDtypeStruct((_B, _NP), jnp.float32),
        mesh=mesh,
        scratch_types=[
            pltpu.VMEM((2, _NHEAD, _NP), jnp.int32),
            pltpu.VMEM((2, _NPAIR, _DIM), jnp.float32),
            pltpu.VMEM((2, _NHEAD, _NP, _DIM), jnp.float32),
            pltpu.VMEM((64, 16), jnp.float32),
            pltpu.VMEM((2, 64), jnp.float32),
            pltpu.SemaphoreType.DMA,
            pltpu.SemaphoreType.DMA,
            pltpu.SemaphoreType.DMA,
            pltpu.SemaphoreType.DMA,
        ],
        compiler_params=pltpu.CompilerParams(
            needs_layout_passes=False, use_tc_tiling_on_sc=False),
    )
    def sc(labels_hbm, gates_hbm, t0, t1, t2, t3, t4, out_hbm,
           lab_v, gate_v, rows_v, accs_v, score_v, sem_g0, sem_g1,
           sem_o0, sem_o1):
        wid = lax.axis_index("s") * _NC + lax.axis_index("c")
        base = wid * _BPW
        last = base + _BPW - 1
        tables = (t0, t1, t2, t3, t4)
        sem_g = (sem_g0, sem_g1)
        sem_o = (sem_o0, sem_o1)
        zero16 = jnp.zeros((16,), jnp.float32)
        lanes = lax.iota(jnp.int32, 16)
        for r in range(_NP, 64):
            accs_v[r] = zero16

        def prefetch(b, s):
            pltpu.sync_copy(labels_hbm.at[b], lab_v.at[s])
            pltpu.async_copy(gates_hbm.at[b], gate_v.at[s], sem_g[s])
            for h in range(_NHEAD):
                pltpu.async_copy(tables[h].at[lab_v.at[s, h]],
                                 rows_v.at[s, h], sem_g[s])

        def drain_g(s):
            pltpu.make_async_copy(gates_hbm.at[0], gate_v.at[s],
                                  sem_g[s]).wait()
            for h in range(_NHEAD):
                pltpu.make_async_copy(labels_hbm.at[0], rows_v.at[s, h],
                                      sem_g[s]).wait()

        def wait_o(s):
            pltpu.make_async_copy(score_v.at[s, pl.ds(0, _NP)],
                                  out_hbm.at[base + s], sem_o[s]).wait()

        def consume(b, s):
            drain_g(s)
            for c in range(4):
                sl = pl.ds(c * 16, 16)
                g0 = gate_v[s, 0, sl]
                g1 = gate_v[s, 1, sl]
                g2 = gate_v[s, 2, sl]
                g3 = gate_v[s, 3, sl]
                g4 = gate_v[s, 4, sl]
                g5 = gate_v[s, 5, sl]

                def neg_body(n, _, c=c, sl=sl, s=s, g0=g0, g1=g1, g2=g2,
                             g3=g3, g4=g4, g5=g5):
                    pred = rows_v[s, 0, n, sl]
                    perm = rows_v[s, 1, n, sl]
                    prim = rows_v[s, 2, n, sl]
                    sec = rows_v[s, 3, n, sl]
                    reo = rows_v[s, 4, n, sl]
                    a1 = pred * g0 + reo * g3
                    a2 = pred * g1 + reo * g4
                    a3 = pred * g2 + reo * g5
                    contrib = a1 * perm + a2 * prim + a3 * sec
                    if c == 0:
                        accs_v[n] = contrib
                    else:
                        plsc.addupdate(accs_v.at[n], contrib)
                    return 0

                lax.fori_loop(0, _NP, neg_body, 0, unroll=8)

            wait_o(s)
            for grp in range(4):
                tot = zero16
                for j in range(16):
                    v = jnp.sum(accs_v[grp * 16 + j])
                    tot = jnp.where(lanes == j, v, tot)
                score_v[s, pl.ds(grp * 16, 16)] = tot
            pltpu.async_copy(score_v.at[s, pl.ds(0, _NP)], out_hbm.at[b],
                             sem_o[s])

        pltpu.async_copy(score_v.at[0, pl.ds(0, _NP)], out_hbm.at[base + 0],
                         sem_o0)
        pltpu.async_copy(score_v.at[1, pl.ds(0, _NP)], out_hbm.at[base + 1],
                         sem_o1)
        prefetch(base, 0)

        def body(k, carry):
            i0 = base + 2 * k
            prefetch(i0 + 1, 1)
            consume(i0, 0)
            prefetch(jnp.minimum(i0 + 2, last), 0)
            consume(i0 + 1, 1)
            return carry

        lax.fori_loop(0, _BPW // 2, body, 0)

        drain_g(0)
        wait_o(0)
        wait_o(1)

    return sc


_sc_kernel = _make_sc_kernel()


def kernel(x, neg_labels, emb_predictor, emb_cf_perm, emb_cf_primary,
           emb_cf_secondary, emb_reorder,
           gw_predictor__cf_perm, gb_predictor__cf_perm,
           gw_predictor__cf_primary, gb_predictor__cf_primary,
           gw_predictor__cf_secondary, gb_predictor__cf_secondary,
           gw_reorder__cf_perm, gb_reorder__cf_perm,
           gw_reorder__cf_primary, gb_reorder__cf_primary,
           gw_reorder__cf_secondary, gb_reorder__cf_secondary):
    gw = jnp.concatenate(
        [gw_predictor__cf_perm, gw_predictor__cf_primary,
         gw_predictor__cf_secondary, gw_reorder__cf_perm,
         gw_reorder__cf_primary, gw_reorder__cf_secondary], axis=1)
    gb = jnp.concatenate(
        [gb_predictor__cf_perm, gb_predictor__cf_primary,
         gb_predictor__cf_secondary, gb_reorder__cf_perm,
         gb_reorder__cf_primary, gb_reorder__cf_secondary], axis=0)
    gates = _gates_tc(x, gw, gb.reshape(1, _NPAIR * _DIM))
    gates = gates.reshape(_B, _NPAIR, _DIM)

    lab = jnp.transpose(neg_labels[:, :, :_NHEAD], (0, 2, 1))
    lab = jnp.pad(lab, ((0, 0), (0, 0), (0, _NP - _NNEG)))

    score = _sc_kernel(lab, gates, emb_predictor, emb_cf_perm,
                       emb_cf_primary, emb_cf_secondary, emb_reorder)
    return score[:, :_NNEG]
